# baseline (speedup 1.0000x reference)
import jax
import jax.numpy as jnp
from jax import lax
from jax.experimental import pallas as pl
from jax.experimental.pallas import tpu as pltpu
from jax.experimental.pallas import tpu_sc as plsc

BATCH = 16384
EMBED_DIM = 10
N_ROWS = 1000000
NUM_CORES = 2
NUM_SUBCORES = 16
LANES = 16
NUM_WORKERS = NUM_CORES * NUM_SUBCORES
B_PER_W = BATCH // NUM_WORKERS
CHUNK = 128
SUBS = B_PER_W // CHUNK
NCHUNK = EMBED_DIM * SUBS
GROUPS = B_PER_W // LANES

DW = 65536
DG = (N_ROWS + DW - 1) // DW


def _detile_body(ut_ref, it_ref, *rest):
    out_refs = rest[:2 * EMBED_DIM]
    sem = rest[2 * EMBED_DIM]
    copies = []
    for d in range(EMBED_DIM):
        copies.append(pltpu.make_async_copy(ut_ref.at[d], out_refs[d], sem))
        copies.append(pltpu.make_async_copy(it_ref.at[d],
                                            out_refs[EMBED_DIM + d], sem))
    for c in copies:
        c.start()
    for c in copies:
        c.wait()


def _detile(ut, it):
    return pl.pallas_call(
        _detile_body,
        grid=(DG,),
        in_specs=[pl.BlockSpec((EMBED_DIM, DW), lambda j: (0, j)),
                  pl.BlockSpec((EMBED_DIM, DW), lambda j: (0, j))],
        out_specs=[pl.BlockSpec((DW,), lambda j: (j,))
                   for _ in range(2 * EMBED_DIM)],
        out_shape=[jax.ShapeDtypeStruct((N_ROWS,), jnp.float32)
                   for _ in range(2 * EMBED_DIM)],
        scratch_shapes=[pltpu.SemaphoreType.DMA],
    )(ut, it)


def _sc_body(users_hbm, items_hbm, *rest):
    u_cols = rest[:EMBED_DIM]
    i_cols = rest[EMBED_DIM:2 * EMBED_DIM]
    out_hbm = rest[2 * EMBED_DIM]
    idx_u, idx_i, u_t, i_t, out_v, sem = rest[2 * EMBED_DIM + 1:]

    wid = lax.axis_index("s") * NUM_CORES + lax.axis_index("c")
    base = wid * B_PER_W

    for sub in range(SUBS):
        pltpu.sync_copy(users_hbm.at[pl.ds(base + sub * CHUNK, CHUNK)],
                        idx_u.at[sub])
        pltpu.sync_copy(items_hbm.at[pl.ds(base + sub * CHUNK, CHUNK)],
                        idx_i.at[sub])

    copies = []
    for d in range(EMBED_DIM):
        for sub in range(SUBS):
            c = d * SUBS + sub
            copies.append(pltpu.make_async_copy(
                u_cols[d].at[idx_u.at[sub]], u_t.at[c], sem))
            copies.append(pltpu.make_async_copy(
                i_cols[d].at[idx_i.at[sub]], i_t.at[c], sem))
    for c in copies:
        c.start()
    for c in copies:
        c.wait()

    for g in range(GROUPS):
        sub = g // (GROUPS // SUBS)
        col = (g % (GROUPS // SUBS)) * LANES
        acc = jnp.zeros((LANES,), jnp.float32)
        for d in range(EMBED_DIM):
            uv = u_t[d * SUBS + sub, pl.ds(col, LANES)]
            iv = i_t[d * SUBS + sub, pl.ds(col, LANES)]
            acc = acc + uv * iv
        sig = 1.0 / (1.0 + jnp.exp(-acc))
        out_v[pl.ds(g * LANES, LANES)] = sig

    pltpu.sync_copy(out_v, out_hbm.at[pl.ds(base, B_PER_W)])


@jax.jit
def _run(users, items, user_embed, item_embed):
    cols = _detile(user_embed.T, item_embed.T)
    u_cols = cols[:EMBED_DIM]
    i_cols = cols[EMBED_DIM:]
    mesh = plsc.VectorSubcoreMesh(core_axis_name="c", subcore_axis_name="s")
    return pl.kernel(
        _sc_body,
        out_type=jax.ShapeDtypeStruct((BATCH,), jnp.float32),
        mesh=mesh,
        compiler_params=pltpu.CompilerParams(needs_layout_passes=False,
                                             use_tc_tiling_on_sc=False),
        scratch_types=[
            pltpu.VMEM((SUBS, CHUNK), jnp.int32),
            pltpu.VMEM((SUBS, CHUNK), jnp.int32),
            pltpu.VMEM((NCHUNK, CHUNK), jnp.float32),
            pltpu.VMEM((NCHUNK, CHUNK), jnp.float32),
            pltpu.VMEM((B_PER_W,), jnp.float32),
            pltpu.SemaphoreType.DMA,
        ],
    )(users, items, *u_cols, *i_cols)


def kernel(users, items, user_embed, item_embed):
    users = users.astype(jnp.int32)
    items = items.astype(jnp.int32)
    return _run(users, items, user_embed, item_embed)


# speedup vs baseline: 1.1163x; 1.1163x over previous
import jax
import jax.numpy as jnp
from jax import lax
from jax.experimental import pallas as pl
from jax.experimental.pallas import tpu as pltpu
from jax.experimental.pallas import tpu_sc as plsc

BATCH = 16384
EMBED_DIM = 10
N_ROWS = 1000000
NUM_CORES = 2
NUM_SUBCORES = 16
LANES = 16
NUM_WORKERS = NUM_CORES * NUM_SUBCORES
B_PER_W = BATCH // NUM_WORKERS
CHUNK = 128
SUBS = B_PER_W // CHUNK
NCHUNK = EMBED_DIM * SUBS
GROUPS = B_PER_W // LANES

DW = 131072
DG = (N_ROWS + DW - 1) // DW


def _detile_body(ut_ref, it_ref, *out_refs):
    for d in range(EMBED_DIM):
        out_refs[d][...] = ut_ref[d, :]
        out_refs[EMBED_DIM + d][...] = it_ref[d, :]


def _detile(ut, it):
    return pl.pallas_call(
        _detile_body,
        grid=(DG,),
        in_specs=[pl.BlockSpec((EMBED_DIM, DW), lambda j: (0, j)),
                  pl.BlockSpec((EMBED_DIM, DW), lambda j: (0, j))],
        out_specs=[pl.BlockSpec((DW,), lambda j: (j,))
                   for _ in range(2 * EMBED_DIM)],
        out_shape=[jax.ShapeDtypeStruct((N_ROWS,), jnp.float32)
                   for _ in range(2 * EMBED_DIM)],
    )(ut, it)


def _sc_body(users_hbm, items_hbm, *rest):
    u_cols = rest[:EMBED_DIM]
    i_cols = rest[EMBED_DIM:2 * EMBED_DIM]
    out_hbm = rest[2 * EMBED_DIM]
    idx_u, idx_i, u_t, i_t, out_v, sem = rest[2 * EMBED_DIM + 1:]

    wid = lax.axis_index("s") * NUM_CORES + lax.axis_index("c")
    base = wid * B_PER_W

    for sub in range(SUBS):
        pltpu.sync_copy(users_hbm.at[pl.ds(base + sub * CHUNK, CHUNK)],
                        idx_u.at[sub])
        pltpu.sync_copy(items_hbm.at[pl.ds(base + sub * CHUNK, CHUNK)],
                        idx_i.at[sub])

    copies = []
    for d in range(EMBED_DIM):
        for sub in range(SUBS):
            c = d * SUBS + sub
            copies.append(pltpu.make_async_copy(
                u_cols[d].at[idx_u.at[sub]], u_t.at[c], sem))
            copies.append(pltpu.make_async_copy(
                i_cols[d].at[idx_i.at[sub]], i_t.at[c], sem))
    for c in copies:
        c.start()
    for c in copies:
        c.wait()

    for g in range(GROUPS):
        sub = g // (GROUPS // SUBS)
        col = (g % (GROUPS // SUBS)) * LANES
        acc = jnp.zeros((LANES,), jnp.float32)
        for d in range(EMBED_DIM):
            uv = u_t[d * SUBS + sub, pl.ds(col, LANES)]
            iv = i_t[d * SUBS + sub, pl.ds(col, LANES)]
            acc = acc + uv * iv
        sig = 1.0 / (1.0 + jnp.exp(-acc))
        out_v[pl.ds(g * LANES, LANES)] = sig

    pltpu.sync_copy(out_v, out_hbm.at[pl.ds(base, B_PER_W)])


@jax.jit
def _run(users, items, user_embed, item_embed):
    cols = _detile(user_embed.T, item_embed.T)
    u_cols = cols[:EMBED_DIM]
    i_cols = cols[EMBED_DIM:]
    mesh = plsc.VectorSubcoreMesh(core_axis_name="c", subcore_axis_name="s")
    return pl.kernel(
        _sc_body,
        out_type=jax.ShapeDtypeStruct((BATCH,), jnp.float32),
        mesh=mesh,
        compiler_params=pltpu.CompilerParams(needs_layout_passes=False,
                                             use_tc_tiling_on_sc=False),
        scratch_types=[
            pltpu.VMEM((SUBS, CHUNK), jnp.int32),
            pltpu.VMEM((SUBS, CHUNK), jnp.int32),
            pltpu.VMEM((NCHUNK, CHUNK), jnp.float32),
            pltpu.VMEM((NCHUNK, CHUNK), jnp.float32),
            pltpu.VMEM((B_PER_W,), jnp.float32),
            pltpu.SemaphoreType.DMA,
        ],
    )(users, items, *u_cols, *i_cols)


def kernel(users, items, user_embed, item_embed):
    users = users.astype(jnp.int32)
    items = items.astype(jnp.int32)
    return _run(users, items, user_embed, item_embed)
